# baseline (speedup 1.0000x reference)
import math

import jax
import jax.numpy as jnp
from jax.experimental import pallas as pl
from jax.experimental.pallas import tpu as pltpu

_BLUR = 0.05
_EPS = _BLUR ** 2
_INV_EPS = 1.0 / _EPS
_N_ITERS = 10


def kernel(y_pred, y_true, length_pred, length_true):
    B, T = y_pred.shape
    Tm2 = T - 2
    BLK = 128
    G = B // BLK
    log_a = math.log(1.0 / T)

    def _body(ypt_ref, ytt_ref, lp_ref, lt_ref, out_ref):
        yp = ypt_ref[1:T - 1, :]
        yt = ytt_ref[1:T - 1, :]
        X = jnp.stack([yt, yt, yp])
        Y = jnp.stack([yp, yt, yp])
        Ce = (0.5 * _INV_EPS) * (X[:, :, None, :] - Y[:, None, :, :]) ** 2
        CeT = (0.5 * _INV_EPS) * (Y[:, :, None, :] - X[:, None, :, :]) ** 2

        def lse_sl(F, Cm):
            outs = []
            for j in range(Tm2):
                v = F - Cm[:, j]
                m = jnp.max(v, axis=1, keepdims=True)
                s = jnp.sum(jnp.exp(v - m), axis=1, keepdims=True)
                outs.append(m + jnp.log(s))
            return jnp.concatenate(outs, axis=1)

        def body_fn(_, carry):
            f, _ = carry
            g = -(lse_sl(f, CeT) + log_a)
            f = -(lse_sl(g, Ce) + log_a)
            return f, g

        zeros = jnp.zeros((3, Tm2, BLK), jnp.float32)
        f, g = jax.lax.fori_loop(0, _N_ITERS, body_fn, (zeros, zeros))
        ot = (jnp.sum(f, axis=1) + jnp.sum(g, axis=1)) * (_EPS / T)
        div = ot[0] - 0.5 * ot[1] - 0.5 * ot[2]
        tim = jnp.sum((yp - yt) ** 2, axis=0)
        dl = lp_ref[0, :] - lt_ref[0, :]
        out_ref[0] = jnp.stack([div, tim, dl * dl])

    out = pl.pallas_call(
        _body,
        grid=(G,),
        in_specs=[
            pl.BlockSpec((T, BLK), lambda i: (0, i)),
            pl.BlockSpec((T, BLK), lambda i: (0, i)),
            pl.BlockSpec((1, BLK), lambda i: (0, i)),
            pl.BlockSpec((1, BLK), lambda i: (0, i)),
        ],
        out_specs=pl.BlockSpec((1, 3, BLK), lambda i: (i, 0, 0)),
        out_shape=jax.ShapeDtypeStruct((G, 3, BLK), jnp.float32),
        compiler_params=pltpu.CompilerParams(
            dimension_semantics=("parallel",)),
    )(y_pred.T, y_true.T, length_pred.reshape(1, B), length_true.reshape(1, B))

    sums = out.sum(axis=(0, 2))
    distrib_loss = sums[0] / B
    timing_loss = sums[1] / (B * Tm2)
    length_loss = sums[2] / B
    weighted_loss = timing_loss + length_loss + distrib_loss
    return (weighted_loss, length_loss, timing_loss)


# speedup vs baseline: 1.2848x; 1.2848x over previous
import math

import jax
import jax.numpy as jnp
from jax.experimental import pallas as pl
from jax.experimental.pallas import tpu as pltpu

_BLUR = 0.05
_EPS = _BLUR ** 2
_INV_EPS = 1.0 / _EPS
_N_ITERS = 10
_LN2 = math.log(2.0)
_L2E = 1.0 / _LN2


def kernel(y_pred, y_true, length_pred, length_true):
    B, T = y_pred.shape
    Tm2 = T - 2
    BLK = 128
    G = B // BLK
    log_a = math.log(1.0 / T)

    def _body(ypt_ref, ytt_ref, lp_ref, lt_ref, out_ref):
        yp = ypt_ref[1:T - 1, :]
        yt = ytt_ref[1:T - 1, :]
        X = jnp.stack([yt, yt, yp])
        Y = jnp.stack([yp, yt, yp])
        c2 = 0.5 * _INV_EPS * _L2E
        Ce = c2 * (X[:, :, None, :] - Y[:, None, :, :]) ** 2
        CeT = c2 * (Y[:, :, None, :] - X[:, None, :, :]) ** 2

        la2 = log_a * _L2E

        def lse1(F, Cm):
            m = F[:, 0, None, :] - Cm[:, 0]
            for i in range(1, Tm2):
                m = jnp.maximum(m, F[:, i, None, :] - Cm[:, i])
            s = jnp.exp2(F[:, 0, None, :] - (Cm[:, 0] + m))
            for i in range(1, Tm2):
                s = s + jnp.exp2(F[:, i, None, :] - (Cm[:, i] + m))
            return m + jnp.log2(s)

        f = jnp.zeros((3, Tm2, BLK), jnp.float32)
        g = f
        for _ in range(_N_ITERS):
            g = -(lse1(f, Ce) + la2)
            f = -(lse1(g, CeT) + la2)
        ot = (jnp.sum(f, axis=1) + jnp.sum(g, axis=1)) * (_EPS * _LN2 / T)
        div = ot[0] - 0.5 * ot[1] - 0.5 * ot[2]
        tim = jnp.sum((yp - yt) ** 2, axis=0)
        dl = lp_ref[0, :] - lt_ref[0, :]
        out_ref[0] = jnp.stack([div, tim, dl * dl])

    out = pl.pallas_call(
        _body,
        grid=(G,),
        in_specs=[
            pl.BlockSpec((T, BLK), lambda i: (0, i)),
            pl.BlockSpec((T, BLK), lambda i: (0, i)),
            pl.BlockSpec((1, BLK), lambda i: (0, i)),
            pl.BlockSpec((1, BLK), lambda i: (0, i)),
        ],
        out_specs=pl.BlockSpec((1, 3, BLK), lambda i: (i, 0, 0)),
        out_shape=jax.ShapeDtypeStruct((G, 3, BLK), jnp.float32),
        compiler_params=pltpu.CompilerParams(
            dimension_semantics=("parallel",)),
    )(y_pred.T, y_true.T, length_pred.reshape(1, B), length_true.reshape(1, B))

    sums = out.sum(axis=(0, 2))
    distrib_loss = sums[0] / B
    timing_loss = sums[1] / (B * Tm2)
    length_loss = sums[2] / B
    weighted_loss = timing_loss + length_loss + distrib_loss
    return (weighted_loss, length_loss, timing_loss)
